# baseline (speedup 1.0000x reference)
import jax
import jax.numpy as jnp
from jax.experimental import pallas as pl
from jax.experimental.pallas import tpu as pltpu

SEQ = 8192
DIM = 4096
NSLAB = 16
ROWS = SEQ // NSLAB


def _body(table_hbm, out_hbm, sems):
    for i in range(NSLAB):
        pltpu.make_async_copy(
            table_hbm.at[pl.ds(i * ROWS, ROWS)],
            out_hbm.at[pl.ds(i * ROWS, ROWS)],
            sems.at[i],
        ).start()
    for i in range(NSLAB):
        pltpu.make_async_copy(
            table_hbm.at[pl.ds(i * ROWS, ROWS)],
            out_hbm.at[pl.ds(i * ROWS, ROWS)],
            sems.at[i],
        ).wait()


_copy = pl.pallas_call(
    _body,
    in_specs=[pl.BlockSpec(memory_space=pltpu.MemorySpace.HBM)],
    out_specs=pl.BlockSpec(memory_space=pltpu.MemorySpace.HBM),
    out_shape=jax.ShapeDtypeStruct((SEQ, DIM), jnp.float32),
    scratch_shapes=[pltpu.SemaphoreType.DMA((NSLAB,))],
)


def kernel(x, emb_weight):
    del x
    return _copy(emb_weight)
